# baseline (speedup 1.0000x reference)
import jax
import jax.numpy as jnp
from jax.experimental import pallas as pl
from jax.experimental.pallas import tpu as pltpu

_B = 64
_K = 16384
_N = 4096
_NB = 256


def _mm_kernel(x_ref, w_ref, b_ref, o_ref):
    xb = x_ref[...].astype(jnp.bfloat16)
    wb = w_ref[...].astype(jnp.bfloat16)
    acc = jax.lax.dot_general(
        xb, wb, (((1,), (1,)), ((), ())),
        preferred_element_type=jnp.float32)
    o_ref[...] = acc + b_ref[...]


def kernel(input, weight, bias):
    bias2 = bias.reshape(1, _N)
    return pl.pallas_call(
        _mm_kernel,
        grid=(_N // _NB,),
        in_specs=[
            pl.BlockSpec((_B, _K), lambda n: (0, 0)),
            pl.BlockSpec((_NB, _K), lambda n: (n, 0)),
            pl.BlockSpec((1, _NB), lambda n: (0, n)),
        ],
        out_specs=pl.BlockSpec((_B, _NB), lambda n: (0, n)),
        out_shape=jax.ShapeDtypeStruct((_B, _N), jnp.float32),
        compiler_params=pltpu.CompilerParams(
            dimension_semantics=("arbitrary",),
        ),
    )(input, weight, bias2)


# speedup vs baseline: 1.0023x; 1.0023x over previous
import jax
import jax.numpy as jnp
from jax.experimental import pallas as pl
from jax.experimental.pallas import tpu as pltpu

_B = 64
_K = 16384
_N = 4096
_NB = 256


def _mm_kernel(x_ref, w_ref, b_ref, o_ref):
    xb = x_ref[...].astype(jnp.bfloat16)
    wb = w_ref[...].astype(jnp.bfloat16)
    acc = jax.lax.dot_general(
        xb, wb, (((1,), (1,)), ((), ())),
        preferred_element_type=jnp.float32)
    o_ref[...] = acc + b_ref[...]


def kernel(input, weight, bias):
    bias2 = bias.reshape(1, _N)
    return pl.pallas_call(
        _mm_kernel,
        grid=(_N // _NB,),
        in_specs=[
            pl.BlockSpec((_B, _K), lambda n: (0, 0)),
            pl.BlockSpec((_NB, _K), lambda n: (n, 0)),
            pl.BlockSpec((1, _NB), lambda n: (0, n)),
        ],
        out_specs=pl.BlockSpec((_B, _NB), lambda n: (0, n)),
        out_shape=jax.ShapeDtypeStruct((_B, _N), jnp.float32),
        compiler_params=pltpu.CompilerParams(
            dimension_semantics=("parallel",),
        ),
    )(input, weight, bias2)


# speedup vs baseline: 1.0331x; 1.0307x over previous
import jax
import jax.numpy as jnp
from jax.experimental import pallas as pl
from jax.experimental.pallas import tpu as pltpu

_B = 64
_K = 16384
_N = 4096
_NB = 256
_NSPLIT = 2
_KS = _K // _NSPLIT


def _mm_kernel(x_ref, b_ref, *w_refs_and_out):
    w_refs = w_refs_and_out[:-1]
    o_ref = w_refs_and_out[-1]
    acc = b_ref[...].astype(jnp.float32)
    for i, w_ref in enumerate(w_refs):
        xb = x_ref[:, i * _KS:(i + 1) * _KS].astype(jnp.bfloat16)
        wb = w_ref[...].astype(jnp.bfloat16)
        acc = acc + jax.lax.dot_general(
            xb, wb, (((1,), (1,)), ((), ())),
            preferred_element_type=jnp.float32)
    o_ref[...] = acc


def kernel(input, weight, bias):
    bias2 = bias.reshape(1, _N)
    w_parts = [weight] * _NSPLIT
    w_specs = [pl.BlockSpec((_NB, _KS), lambda n, i=i: (n, i))
               for i in range(_NSPLIT)]
    return pl.pallas_call(
        _mm_kernel,
        grid=(_N // _NB,),
        in_specs=[
            pl.BlockSpec((_B, _K), lambda n: (0, 0)),
            pl.BlockSpec((1, _NB), lambda n: (0, n)),
        ] + w_specs,
        out_specs=pl.BlockSpec((_B, _NB), lambda n: (0, n)),
        out_shape=jax.ShapeDtypeStruct((_B, _N), jnp.float32),
        compiler_params=pltpu.CompilerParams(
            dimension_semantics=("arbitrary",),
        ),
    )(input, bias2, *w_parts)
